# baseline (speedup 1.0000x reference)
import functools

import jax
import jax.numpy as jnp
from jax.experimental import pallas as pl
from jax.experimental.pallas import tpu as pltpu

_D = 64
_BLK = 4096


def _pad_rows(x, m):
    p = (-x.shape[0]) % m
    if p:
        x = jnp.pad(x, ((0, p), (0, 0)))
    return x


def _mlp(x1, w1, b1, w3, b3, x2=None, w2=None, adds=()):
    n0 = x1.shape[0]
    x1p = _pad_rows(x1, _BLK)
    npad = x1p.shape[0]
    grid = npad // _BLK

    row_spec = lambda c: pl.BlockSpec((_BLK, c), lambda i: (i, 0))
    full_spec = lambda a: pl.BlockSpec(a.shape, lambda i: (0,) * a.ndim)

    args = [x1p, w1, b1.reshape(1, -1)]
    specs = [row_spec(x1p.shape[1]), full_spec(w1), full_spec(b1.reshape(1, -1))]
    has_x2 = x2 is not None
    if has_x2:
        x2p = _pad_rows(x2, _BLK)
        args += [x2p, w2]
        specs += [row_spec(x2p.shape[1]), full_spec(w2)]
    args += [w3, b3.reshape(1, -1)]
    specs += [full_spec(w3), full_spec(b3.reshape(1, -1))]
    adds_p = tuple(_pad_rows(a, _BLK) for a in adds)
    for a in adds_p:
        args.append(a)
        specs.append(row_spec(a.shape[1]))
    dout = w3.shape[1]

    def body(*refs):
        x1r, w1r, b1r = refs[0], refs[1], refs[2]
        i = 3
        pre = jnp.dot(x1r[...], w1r[...], preferred_element_type=jnp.float32)
        if has_x2:
            x2r, w2r = refs[i], refs[i + 1]
            i += 2
            pre = pre + jnp.dot(x2r[...], w2r[...],
                                preferred_element_type=jnp.float32)
        pre = pre + b1r[...]
        h = jnp.maximum(pre, 0.0)
        w3r, b3r = refs[i], refs[i + 1]
        i += 2
        o = jnp.dot(h, w3r[...], preferred_element_type=jnp.float32) + b3r[...]
        for k in range(len(adds_p)):
            o = o + refs[i + k][...]
        refs[-1][...] = o

    out = pl.pallas_call(
        body,
        grid=(grid,),
        in_specs=specs,
        out_specs=pl.BlockSpec((_BLK, dout), lambda i: (i, 0)),
        out_shape=jax.ShapeDtypeStruct((npad, dout), jnp.float32),
    )(*args)
    return out[:n0]


def _linear(x, w, b):
    n0 = x.shape[0]
    xp = _pad_rows(x, _BLK)
    grid = xp.shape[0] // _BLK
    dout = w.shape[1]

    def body(xr, wr, br, outr):
        outr[...] = jnp.dot(xr[...], wr[...],
                            preferred_element_type=jnp.float32) + br[...]

    out = pl.pallas_call(
        body,
        grid=(grid,),
        in_specs=[
            pl.BlockSpec((_BLK, x.shape[1]), lambda i: (i, 0)),
            pl.BlockSpec(w.shape, lambda i: (0, 0)),
            pl.BlockSpec((1, dout), lambda i: (0, 0)),
        ],
        out_specs=pl.BlockSpec((_BLK, dout), lambda i: (i, 0)),
        out_shape=jax.ShapeDtypeStruct((xp.shape[0], dout), jnp.float32),
    )(xp, w, b.reshape(1, -1))
    return out[:n0]


def _pad8(x):
    return jnp.pad(x, ((0, 0), (0, 8 - x.shape[1])))


def kernel(remission, points, l1_cluster_centers, l2_cluster_centers,
           l1_edges, l2_edges, l1_labels, l2_labels,
           Wf1, bf1, Wf2, bf2,
           We1, be1, We2, be2, Wo1, bo1, Wo2, bo2,
           Wc, bc):
    f32 = jnp.float32
    n1 = l1_cluster_centers.shape[0]
    n2 = l2_cluster_centers.shape[0]

    Wf1p = jnp.pad(Wf1, ((0, 4), (0, 0)))
    We1a = We1[:, :_D, :]
    We1b = jnp.pad(We1[:, _D:, :], ((0, 0), (0, 5), (0, 0)))

    pin = _pad8(jnp.concatenate(
        [remission, points - l1_cluster_centers[l1_labels]], axis=1))
    h0 = _mlp(pin, Wf1p, bf1, Wf2, bf2)
    t1 = jnp.zeros((n1, _D), f32).at[l1_labels].add(h0)

    def gnn(feat, centers, edges, i, adds=()):
        src, dst = edges[0], edges[1]
        g = feat[src]
        r = _pad8(centers[src] - centers[dst])
        hh = _mlp(g, We1a[i], be1[i], We2[i], be2[i], x2=r, w2=We1b[i])
        agg = jnp.zeros((feat.shape[0], _D), f32).at[dst].max(hh)
        return _mlp(agg, Wo1[i], bo1[i], Wo2[i], bo2[i], adds=(feat,) + adds)

    t2 = gnn(t1, l1_cluster_centers, l1_edges, 0)
    t2_1 = gnn(t2, l1_cluster_centers, l1_edges, 1)
    t2_2 = gnn(t2_1, l1_cluster_centers, l1_edges, 2)

    t3 = jnp.zeros((n2, _D), f32).at[l2_labels].max(t2_2)
    t4 = t3 + jnp.zeros((n2, _D), f32).at[l2_edges[1]].max(t3[l2_edges[0]])
    t5 = t4[l2_labels]

    t6 = gnn(t5, l1_cluster_centers, l1_edges, 3, adds=(t2_2,))
    t6 = gnn(t6, l1_cluster_centers, l1_edges, 4, adds=(t2_1,))
    t6 = gnn(t6, l1_cluster_centers, l1_edges, 5, adds=(t2,))

    logits_c = _linear(t6, Wc, bc)
    return logits_c[l1_labels]
